# Initial kernel scaffold; baseline (speedup 1.0000x reference)
import jax
import jax.numpy as jnp
from jax.experimental import pallas as pl


def kernel(x, edge_index, W1, g0, b0, Wl1, Wr1, att1, g1, b1, W2, c2, Wl2, Wr2, att2, g2, bb2, W3, c3, Wl3, Wr3, att3, g3, bb3, Wp, cp):
    raise NotImplementedError("write your pallas kernel here")


# speedup vs baseline: 33.1604x; 33.1604x over previous
import functools

import jax
import jax.numpy as jnp
import numpy as np
from jax import lax
from jax.experimental import pallas as pl
from jax.experimental.pallas import tpu as pltpu
from jax.experimental.pallas import tpu_sc as plsc

N = 10000
E = 320000
ETOT = E + N
NW = 32
CH = 128
STEPS = 81
EW = STEPS * CH
EP = NW * EW
EBLK = 4096
NACC = 10240
ROWS_T = NACC // 16

_f32 = jnp.float32


def _mesh():
    return plsc.VectorSubcoreMesh(core_axis_name="c", subcore_axis_name="s",
                                  num_cores=2, num_subcores=16)


_SC_PARAMS = pltpu.CompilerParams(use_tc_tiling_on_sc=False)


@functools.lru_cache(maxsize=None)
def _make_sc_gather(hcp):
    @functools.partial(
        pl.kernel,
        out_type=(
            jax.ShapeDtypeStruct((EP, hcp), _f32),
            jax.ShapeDtypeStruct((EP, hcp), _f32),
        ),
        mesh=_mesh(),
        scratch_types=[
            pltpu.VMEM((STEPS, CH), jnp.int32),
            pltpu.VMEM((STEPS, CH), jnp.int32),
            pltpu.VMEM((CH, hcp), _f32),
            pltpu.VMEM((CH, hcp), _f32),
            pltpu.SemaphoreType.DMA,
            pltpu.SemaphoreType.DMA,
        ],
        compiler_params=_SC_PARAMS,
    )
    def k(xl_hbm, xr_hbm, src_hbm, dst_hbm, gxl_hbm, gxr_hbm,
          sidx, didx, lbuf, rbuf, lsem, rsem):
        wid = lax.axis_index("s") * 2 + lax.axis_index("c")
        base = wid * EW
        pltpu.sync_copy(src_hbm.at[wid], sidx)
        pltpu.sync_copy(dst_hbm.at[wid], didx)

        def body(g, carry):
            cl = pltpu.async_copy(xl_hbm.at[sidx.at[g]], lbuf, lsem)
            cr = pltpu.async_copy(xr_hbm.at[didx.at[g]], rbuf, rsem)
            cl.wait()
            cr.wait()
            pltpu.sync_copy(lbuf, gxl_hbm.at[pl.ds(base + g * CH, CH)])
            pltpu.sync_copy(rbuf, gxr_hbm.at[pl.ds(base + g * CH, CH)])
            return carry

        lax.fori_loop(0, STEPS, body, 0)

    return k


@functools.lru_cache(maxsize=None)
def _make_sc_scatter(w):
    @functools.partial(
        pl.kernel,
        out_type=jax.ShapeDtypeStruct((2, NACC, w), _f32),
        mesh=_mesh(),
        scratch_types=[
            pltpu.VMEM((STEPS, CH), jnp.int32),
            pltpu.VMEM((CH, w), _f32),
            pltpu.VMEM_SHARED((NACC, w), _f32),
        ],
        compiler_params=_SC_PARAMS,
    )
    def k(s_hbm, dst_hbm, zero_hbm, out_hbm, didx, sbuf, acc):
        c = lax.axis_index("c")
        s = lax.axis_index("s")
        wid = s * 2 + c
        base = wid * EW
        pltpu.sync_copy(zero_hbm, acc.at[pl.ds(s * ROWS_T, ROWS_T)])
        pltpu.sync_copy(dst_hbm.at[wid], didx)
        plsc.subcore_barrier()

        def body(g, carry):
            pltpu.sync_copy(s_hbm.at[pl.ds(base + g * CH, CH)], sbuf)
            pltpu.sync_copy(sbuf, acc.at[didx.at[g]], add=True)
            return carry

        lax.fori_loop(0, STEPS, body, 0)
        plsc.subcore_barrier()
        pltpu.sync_copy(acc.at[pl.ds(s * ROWS_T, ROWS_T)],
                        out_hbm.at[c, pl.ds(s * ROWS_T, ROWS_T)])

    return k


def _elu(x):
    return jnp.where(x > 0, x, jnp.exp(jnp.minimum(x, 0.0)) - 1.0)


def _bn(x, g, b):
    m = jnp.mean(x, axis=0, keepdims=True)
    v = jnp.mean((x - m) ** 2, axis=0, keepdims=True)
    return (x - m) / jnp.sqrt(v + 1e-5) * g + b


def _call_tc(body, out_shapes, args):
    return pl.pallas_call(
        body,
        out_shape=out_shapes,
    )(*args)


def _tc_dense1(x, W1, g0, b0, Wl1, Wr1):
    def body(x_r, w1_r, g0_r, b0_r, wl_r, wr_r, xl_o, xr_o):
        h = jnp.dot(x_r[...], w1_r[...], preferred_element_type=_f32)
        h = _bn(_elu(h), g0_r[...], b0_r[...])
        xl_o[...] = jnp.dot(h, wl_r[...], preferred_element_type=_f32)
        xr_o[...] = jnp.dot(h, wr_r[...], preferred_element_type=_f32)

    return _call_tc(
        body,
        (jax.ShapeDtypeStruct((N, 64), _f32),
         jax.ShapeDtypeStruct((N, 64), _f32)),
        (x, W1, g0.reshape(1, -1), b0.reshape(1, -1), Wl1, Wr1),
    )


def _tc_edge(gxl, gxr, attp, bsum, bbc, w):
    hcp = gxl.shape[1]
    hp = bsum.shape[1]

    def body(xl_r, xr_r, att_r, bs_r, bb_r, s_o):
        e = xl_r[...] + xr_r[...]
        e = jnp.where(e > 0, e, 0.2 * e)
        alpha = jnp.dot(e * att_r[...], bs_r[...],
                        preferred_element_type=_f32, precision=jax.lax.Precision.HIGHEST)
        ex = jnp.exp(alpha)
        msg = jnp.dot(ex, bb_r[...], preferred_element_type=_f32, precision=jax.lax.Precision.HIGHEST) * xl_r[...]
        s_o[...] = jnp.concatenate(
            [msg, ex, jnp.zeros((EBLK, w - hcp - hp), _f32)], axis=1)

    grid = EP // EBLK
    return pl.pallas_call(
        body,
        grid=(grid,),
        in_specs=[
            pl.BlockSpec((EBLK, hcp), lambda i: (i, 0)),
            pl.BlockSpec((EBLK, hcp), lambda i: (i, 0)),
            pl.BlockSpec((1, hcp), lambda i: (0, 0)),
            pl.BlockSpec((hcp, hp), lambda i: (0, 0)),
            pl.BlockSpec((hp, hcp), lambda i: (0, 0)),
        ],
        out_specs=pl.BlockSpec((EBLK, w), lambda i: (i, 0)),
        out_shape=jax.ShapeDtypeStruct((EP, w), _f32),
    )(gxl, gxr, attp, bsum, bbc)


def _tc_dense2(parts, g1, b1, W2, c2, Wl2, Wr2, bbc8, pad2):
    def body2(p_r, g1_r, b1_r, w2_r, c2_r, wl_r, wr_r, bb_r, pad_r, xl_o, xr_o):
        p = p_r[0] + p_r[1]
        num = p[:N, :64]
        den = p[:N, 64:72] + 1e-16
        a = num / jnp.dot(den, bb_r[...], preferred_element_type=_f32, precision=jax.lax.Precision.HIGHEST)
        h = _bn(a, g1_r[...], b1_r[...])
        h = _elu(jnp.dot(h, w2_r[...], preferred_element_type=_f32)
                 + c2_r[...])
        xl = jnp.dot(h, wl_r[...], preferred_element_type=_f32)
        xr = jnp.dot(h, wr_r[...], preferred_element_type=_f32)
        xl_o[...] = jnp.dot(xl, pad_r[...], preferred_element_type=_f32, precision=jax.lax.Precision.HIGHEST)
        xr_o[...] = jnp.dot(xr, pad_r[...], preferred_element_type=_f32, precision=jax.lax.Precision.HIGHEST)

    return _call_tc(
        body2,
        (jax.ShapeDtypeStruct((N, 64), _f32),
         jax.ShapeDtypeStruct((N, 64), _f32)),
        (parts, g1.reshape(1, -1), b1.reshape(1, -1), W2,
         c2.reshape(1, -1), Wl2, Wr2, bbc8, pad2),
    )


def _tc_dense3(parts, sel2, bbc8, g2, bb2, W3, c3, Wl3, Wr3, pad3):
    def body(p_r, sel_r, bb_r, g2_r, b2_r, w3_r, c3_r, wl_r, wr_r, pad_r,
             xl_o, xr_o):
        p = p_r[0] + p_r[1]
        num = p[:N, :64]
        den = p[:N, 64:72] + 1e-16
        a = num / jnp.dot(den, bb_r[...], preferred_element_type=_f32, precision=jax.lax.Precision.HIGHEST)
        g48 = jnp.dot(a, sel_r[...], preferred_element_type=_f32, precision=jax.lax.Precision.HIGHEST)
        h = _bn(g48, g2_r[...], b2_r[...])
        h = _elu(jnp.dot(h, w3_r[...], preferred_element_type=_f32)
                 + c3_r[...])
        xl = jnp.dot(h, wl_r[...], preferred_element_type=_f32)
        xr = jnp.dot(h, wr_r[...], preferred_element_type=_f32)
        xl_o[...] = jnp.dot(xl, pad_r[...], preferred_element_type=_f32, precision=jax.lax.Precision.HIGHEST)
        xr_o[...] = jnp.dot(xr, pad_r[...], preferred_element_type=_f32, precision=jax.lax.Precision.HIGHEST)

    return _call_tc(
        body,
        (jax.ShapeDtypeStruct((N, 48), _f32),
         jax.ShapeDtypeStruct((N, 48), _f32)),
        (parts, sel2, bbc8, g2.reshape(1, -1), bb2.reshape(1, -1), W3,
         c3.reshape(1, -1), Wl3, Wr3, pad3),
    )


def _tc_dense4(parts, sel3, bbc6, g3, bb3, Wp, cp):
    def body(p_r, sel_r, bb_r, g3_r, b3_r, wp_r, cp_r, out_o):
        p = p_r[0] + p_r[1]
        num = p[:N, :48]
        den = p[:N, 48:54] + 1e-16
        a = num / jnp.dot(den, bb_r[...], preferred_element_type=_f32, precision=jax.lax.Precision.HIGHEST)
        g25 = jnp.dot(a, sel_r[...], preferred_element_type=_f32, precision=jax.lax.Precision.HIGHEST)
        h = _bn(g25, g3_r[...], b3_r[...])
        out_o[...] = jnp.dot(h, wp_r[...], preferred_element_type=_f32) \
            + cp_r[...]

    return _call_tc(
        body,
        jax.ShapeDtypeStruct((N, 2), _f32),
        (parts, sel3, bbc6, g3.reshape(1, -1), bb3.reshape(1, -1), Wp,
         cp.reshape(1, -1)),
    )


def _head_sum(hp, hcp):
    m = np.zeros((hcp, hp), np.float32)
    for h in range(hp):
        m[8 * h:8 * h + 8, h] = 1.0
    return m


def _head_bcast(hp, hcp):
    return np.ascontiguousarray(np.transpose(_head_sum(hp, hcp)))


def _pad_mat(h, c, hp):
    m = np.zeros((h * c, hp * 8), np.float32)
    for hh in range(h):
        for cc in range(c):
            m[hh * c + cc, 8 * hh + cc] = 1.0
    return m


def _sel_mat(h, c, hp):
    return np.ascontiguousarray(np.transpose(_pad_mat(h, c, hp)))


_BSUM8 = _head_sum(8, 64)
_BBC8 = _head_bcast(8, 64)
_BSUM6 = _head_sum(6, 48)
_BBC6 = _head_bcast(6, 48)
_PAD2 = _pad_mat(8, 6, 8)
_SEL2 = _sel_mat(8, 6, 8)
_PAD3 = _pad_mat(5, 5, 6)
_SEL3 = _sel_mat(5, 5, 6)
_Z80 = np.zeros((ROWS_T, 80), np.float32)
_Z64 = np.zeros((ROWS_T, 64), np.float32)


def kernel(x, edge_index, W1, g0, b0, Wl1, Wr1, att1, g1, b1, W2, c2, Wl2,
           Wr2, att2, g2, bb2, W3, c3, Wl3, Wr3, att3, g3, bb3, Wp, cp):
    idt = jnp.int32
    loops = jnp.arange(N, dtype=idt)
    src = jnp.concatenate([edge_index[0].astype(idt), loops])
    dst = jnp.concatenate([edge_index[1].astype(idt), loops])
    npad = EP - ETOT
    srcg = jnp.concatenate([src, jnp.zeros((npad,), idt)]).reshape(
        NW, STEPS, CH)
    dstg = jnp.concatenate([dst, jnp.zeros((npad,), idt)]).reshape(
        NW, STEPS, CH)
    dsts = jnp.concatenate([dst, jnp.full((npad,), N, idt)]).reshape(
        NW, STEPS, CH)

    att1p = att1.reshape(1, 64)
    att2p = jnp.concatenate([att2, jnp.zeros((8, 2), _f32)],
                            axis=1).reshape(1, 64)
    att3p = jnp.concatenate(
        [jnp.concatenate([att3, jnp.zeros((5, 3), _f32)], axis=1),
         jnp.zeros((1, 8), _f32)], axis=0).reshape(1, 48)

    xl1, xr1 = _tc_dense1(x, W1, g0, b0, Wl1, Wr1)
    gxl1, gxr1 = _make_sc_gather(64)(xl1, xr1, srcg, dstg)
    s1 = _tc_edge(gxl1, gxr1, att1p, _BSUM8, _BBC8, 80)
    p1 = _make_sc_scatter(80)(s1, dsts, _Z80)

    xl2, xr2 = _tc_dense2(p1, g1, b1, W2, c2, Wl2, Wr2, _BBC8, _PAD2)
    gxl2, gxr2 = _make_sc_gather(64)(xl2, xr2, srcg, dstg)
    s2 = _tc_edge(gxl2, gxr2, att2p, _BSUM8, _BBC8, 80)
    p2 = _make_sc_scatter(80)(s2, dsts, _Z80)

    xl3, xr3 = _tc_dense3(p2, _SEL2, _BBC8, g2, bb2, W3, c3, Wl3, Wr3, _PAD3)
    gxl3, gxr3 = _make_sc_gather(48)(xl3, xr3, srcg, dstg)
    s3 = _tc_edge(gxl3, gxr3, att3p, _BSUM6, _BBC6, 64)
    p3 = _make_sc_scatter(64)(s3, dsts, _Z64)

    return _tc_dense4(p3, _SEL3, _BBC6, g3, bb3, Wp, cp)


# speedup vs baseline: 35.9190x; 1.0832x over previous
import functools

import jax
import jax.numpy as jnp
import numpy as np
from jax import lax
from jax.experimental import pallas as pl
from jax.experimental.pallas import tpu as pltpu
from jax.experimental.pallas import tpu_sc as plsc

N = 10000
E = 320000
ETOT = E + N
NW = 32
CH = 128
STEPS = 81
EW = STEPS * CH
EP = NW * EW
EBLK = 4096
NACC = 10240
ROWS_T = NACC // 16

_f32 = jnp.float32


def _mesh():
    return plsc.VectorSubcoreMesh(core_axis_name="c", subcore_axis_name="s",
                                  num_cores=2, num_subcores=16)


_SC_PARAMS = pltpu.CompilerParams(use_tc_tiling_on_sc=False)


@functools.lru_cache(maxsize=None)
def _make_sc_gather(hcp):
    @functools.partial(
        pl.kernel,
        out_type=(
            jax.ShapeDtypeStruct((EP, hcp), _f32),
            jax.ShapeDtypeStruct((EP, hcp), _f32),
        ),
        mesh=_mesh(),
        scratch_types=[
            pltpu.VMEM((STEPS, CH), jnp.int32),
            pltpu.VMEM((STEPS, CH), jnp.int32),
            pltpu.VMEM((CH, hcp), _f32),
            pltpu.VMEM((CH, hcp), _f32),
            pltpu.VMEM((CH, hcp), _f32),
            pltpu.VMEM((CH, hcp), _f32),
            pltpu.SemaphoreType.DMA,
            pltpu.SemaphoreType.DMA,
        ],
        compiler_params=_SC_PARAMS,
    )
    def k(xl_hbm, xr_hbm, src_hbm, dst_hbm, gxl_hbm, gxr_hbm,
          sidx, didx, lbuf0, rbuf0, lbuf1, rbuf1, lsem, rsem):
        wid = lax.axis_index("s") * 2 + lax.axis_index("c")
        base = wid * EW
        pltpu.sync_copy(src_hbm.at[wid], sidx)
        pltpu.sync_copy(dst_hbm.at[wid], didx)
        lbufs = (lbuf0, lbuf1)
        rbufs = (rbuf0, rbuf1)

        pltpu.async_copy(xl_hbm.at[sidx.at[0]], lbuf0, lsem)
        pltpu.async_copy(xr_hbm.at[didx.at[0]], rbuf0, rsem)

        def body(i, carry):
            for b in range(2):
                g = 2 * i + b
                nb = 1 - b

                @pl.when(g + 1 < STEPS)
                def _():
                    pltpu.async_copy(xl_hbm.at[sidx.at[g + 1]], lbufs[nb],
                                     lsem)
                    pltpu.async_copy(xr_hbm.at[didx.at[g + 1]], rbufs[nb],
                                     rsem)

                pltpu.make_async_copy(xl_hbm.at[sidx.at[g]], lbufs[b],
                                      lsem).wait()
                pltpu.make_async_copy(xr_hbm.at[didx.at[g]], rbufs[b],
                                      rsem).wait()
                pltpu.sync_copy(lbufs[b], gxl_hbm.at[pl.ds(base + g * CH,
                                                           CH)])
                pltpu.sync_copy(rbufs[b], gxr_hbm.at[pl.ds(base + g * CH,
                                                           CH)])
            return carry

        lax.fori_loop(0, (STEPS - 1) // 2, body, 0)
        g = STEPS - 1
        pltpu.make_async_copy(xl_hbm.at[sidx.at[g]], lbuf0, lsem).wait()
        pltpu.make_async_copy(xr_hbm.at[didx.at[g]], rbuf0, rsem).wait()
        pltpu.sync_copy(lbuf0, gxl_hbm.at[pl.ds(base + g * CH, CH)])
        pltpu.sync_copy(rbuf0, gxr_hbm.at[pl.ds(base + g * CH, CH)])

    return k


@functools.lru_cache(maxsize=None)
def _make_sc_scatter(w):
    @functools.partial(
        pl.kernel,
        out_type=jax.ShapeDtypeStruct((2, NACC, w), _f32),
        mesh=_mesh(),
        scratch_types=[
            pltpu.VMEM((STEPS, CH), jnp.int32),
            pltpu.VMEM((CH, w), _f32),
            pltpu.VMEM((CH, w), _f32),
            pltpu.VMEM_SHARED((NACC, w), _f32),
            pltpu.SemaphoreType.DMA,
        ],
        compiler_params=_SC_PARAMS,
    )
    def k(s_hbm, dst_hbm, zero_hbm, out_hbm, didx, sbuf0, sbuf1, acc, ssem):
        c = lax.axis_index("c")
        s = lax.axis_index("s")
        wid = s * 2 + c
        base = wid * EW
        pltpu.sync_copy(zero_hbm, acc.at[pl.ds(s * ROWS_T, ROWS_T)])
        pltpu.sync_copy(dst_hbm.at[wid], didx)
        plsc.subcore_barrier()
        sbufs = (sbuf0, sbuf1)

        pltpu.async_copy(s_hbm.at[pl.ds(base, CH)], sbuf0, ssem)

        def body(i, carry):
            for b in range(2):
                g = 2 * i + b
                nb = 1 - b

                @pl.when(g + 1 < STEPS)
                def _():
                    pltpu.async_copy(
                        s_hbm.at[pl.ds(base + (g + 1) * CH, CH)],
                        sbufs[nb], ssem)

                pltpu.make_async_copy(s_hbm.at[pl.ds(base + g * CH, CH)],
                                      sbufs[b], ssem).wait()
                pltpu.sync_copy(sbufs[b], acc.at[didx.at[g]], add=True)
            return carry

        lax.fori_loop(0, (STEPS - 1) // 2, body, 0)
        g = STEPS - 1
        pltpu.make_async_copy(s_hbm.at[pl.ds(base + g * CH, CH)],
                              sbuf0, ssem).wait()
        pltpu.sync_copy(sbuf0, acc.at[didx.at[g]], add=True)
        plsc.subcore_barrier()
        pltpu.sync_copy(acc.at[pl.ds(s * ROWS_T, ROWS_T)],
                        out_hbm.at[c, pl.ds(s * ROWS_T, ROWS_T)])

    return k


def _elu(x):
    return jnp.where(x > 0, x, jnp.exp(jnp.minimum(x, 0.0)) - 1.0)


def _bn(x, g, b):
    m = jnp.mean(x, axis=0, keepdims=True)
    v = jnp.mean((x - m) ** 2, axis=0, keepdims=True)
    return (x - m) / jnp.sqrt(v + 1e-5) * g + b


def _call_tc(body, out_shapes, args):
    return pl.pallas_call(
        body,
        out_shape=out_shapes,
    )(*args)


def _tc_dense1(x, W1, g0, b0, Wl1, Wr1):
    def body(x_r, w1_r, g0_r, b0_r, wl_r, wr_r, xl_o, xr_o):
        h = jnp.dot(x_r[...], w1_r[...], preferred_element_type=_f32)
        h = _bn(_elu(h), g0_r[...], b0_r[...])
        xl_o[...] = jnp.dot(h, wl_r[...], preferred_element_type=_f32)
        xr_o[...] = jnp.dot(h, wr_r[...], preferred_element_type=_f32)

    return _call_tc(
        body,
        (jax.ShapeDtypeStruct((N, 64), _f32),
         jax.ShapeDtypeStruct((N, 64), _f32)),
        (x, W1, g0.reshape(1, -1), b0.reshape(1, -1), Wl1, Wr1),
    )


def _tc_edge(gxl, gxr, attp, bsum, bbc, w):
    hcp = gxl.shape[1]
    hp = bsum.shape[1]

    def body(xl_r, xr_r, att_r, bs_r, bb_r, s_o):
        e = xl_r[...] + xr_r[...]
        e = jnp.where(e > 0, e, 0.2 * e)
        alpha = jnp.dot(e * att_r[...], bs_r[...],
                        preferred_element_type=_f32, precision=jax.lax.Precision.HIGHEST)
        ex = jnp.exp(alpha)
        msg = jnp.dot(ex, bb_r[...], preferred_element_type=_f32, precision=jax.lax.Precision.HIGHEST) * xl_r[...]
        s_o[...] = jnp.concatenate(
            [msg, ex, jnp.zeros((EBLK, w - hcp - hp), _f32)], axis=1)

    grid = EP // EBLK
    return pl.pallas_call(
        body,
        grid=(grid,),
        in_specs=[
            pl.BlockSpec((EBLK, hcp), lambda i: (i, 0)),
            pl.BlockSpec((EBLK, hcp), lambda i: (i, 0)),
            pl.BlockSpec((1, hcp), lambda i: (0, 0)),
            pl.BlockSpec((hcp, hp), lambda i: (0, 0)),
            pl.BlockSpec((hp, hcp), lambda i: (0, 0)),
        ],
        out_specs=pl.BlockSpec((EBLK, w), lambda i: (i, 0)),
        out_shape=jax.ShapeDtypeStruct((EP, w), _f32),
    )(gxl, gxr, attp, bsum, bbc)


def _tc_dense2(parts, g1, b1, W2, c2, Wl2, Wr2, bbc8, pad2):
    def body2(p_r, g1_r, b1_r, w2_r, c2_r, wl_r, wr_r, bb_r, pad_r, xl_o, xr_o):
        p = p_r[0] + p_r[1]
        num = p[:N, :64]
        den = p[:N, 64:72] + 1e-16
        a = num / jnp.dot(den, bb_r[...], preferred_element_type=_f32, precision=jax.lax.Precision.HIGHEST)
        h = _bn(a, g1_r[...], b1_r[...])
        h = _elu(jnp.dot(h, w2_r[...], preferred_element_type=_f32)
                 + c2_r[...])
        xl = jnp.dot(h, wl_r[...], preferred_element_type=_f32)
        xr = jnp.dot(h, wr_r[...], preferred_element_type=_f32)
        xl_o[...] = jnp.dot(xl, pad_r[...], preferred_element_type=_f32, precision=jax.lax.Precision.HIGHEST)
        xr_o[...] = jnp.dot(xr, pad_r[...], preferred_element_type=_f32, precision=jax.lax.Precision.HIGHEST)

    return _call_tc(
        body2,
        (jax.ShapeDtypeStruct((N, 64), _f32),
         jax.ShapeDtypeStruct((N, 64), _f32)),
        (parts, g1.reshape(1, -1), b1.reshape(1, -1), W2,
         c2.reshape(1, -1), Wl2, Wr2, bbc8, pad2),
    )


def _tc_dense3(parts, sel2, bbc8, g2, bb2, W3, c3, Wl3, Wr3, pad3):
    def body(p_r, sel_r, bb_r, g2_r, b2_r, w3_r, c3_r, wl_r, wr_r, pad_r,
             xl_o, xr_o):
        p = p_r[0] + p_r[1]
        num = p[:N, :64]
        den = p[:N, 64:72] + 1e-16
        a = num / jnp.dot(den, bb_r[...], preferred_element_type=_f32, precision=jax.lax.Precision.HIGHEST)
        g48 = jnp.dot(a, sel_r[...], preferred_element_type=_f32, precision=jax.lax.Precision.HIGHEST)
        h = _bn(g48, g2_r[...], b2_r[...])
        h = _elu(jnp.dot(h, w3_r[...], preferred_element_type=_f32)
                 + c3_r[...])
        xl = jnp.dot(h, wl_r[...], preferred_element_type=_f32)
        xr = jnp.dot(h, wr_r[...], preferred_element_type=_f32)
        xl_o[...] = jnp.dot(xl, pad_r[...], preferred_element_type=_f32, precision=jax.lax.Precision.HIGHEST)
        xr_o[...] = jnp.dot(xr, pad_r[...], preferred_element_type=_f32, precision=jax.lax.Precision.HIGHEST)

    return _call_tc(
        body,
        (jax.ShapeDtypeStruct((N, 48), _f32),
         jax.ShapeDtypeStruct((N, 48), _f32)),
        (parts, sel2, bbc8, g2.reshape(1, -1), bb2.reshape(1, -1), W3,
         c3.reshape(1, -1), Wl3, Wr3, pad3),
    )


def _tc_dense4(parts, sel3, bbc6, g3, bb3, Wp, cp):
    def body(p_r, sel_r, bb_r, g3_r, b3_r, wp_r, cp_r, out_o):
        p = p_r[0] + p_r[1]
        num = p[:N, :48]
        den = p[:N, 48:54] + 1e-16
        a = num / jnp.dot(den, bb_r[...], preferred_element_type=_f32, precision=jax.lax.Precision.HIGHEST)
        g25 = jnp.dot(a, sel_r[...], preferred_element_type=_f32, precision=jax.lax.Precision.HIGHEST)
        h = _bn(g25, g3_r[...], b3_r[...])
        out_o[...] = jnp.dot(h, wp_r[...], preferred_element_type=_f32) \
            + cp_r[...]

    return _call_tc(
        body,
        jax.ShapeDtypeStruct((N, 2), _f32),
        (parts, sel3, bbc6, g3.reshape(1, -1), bb3.reshape(1, -1), Wp,
         cp.reshape(1, -1)),
    )


def _head_sum(hp, hcp):
    m = np.zeros((hcp, hp), np.float32)
    for h in range(hp):
        m[8 * h:8 * h + 8, h] = 1.0
    return m


def _head_bcast(hp, hcp):
    return np.ascontiguousarray(np.transpose(_head_sum(hp, hcp)))


def _pad_mat(h, c, hp):
    m = np.zeros((h * c, hp * 8), np.float32)
    for hh in range(h):
        for cc in range(c):
            m[hh * c + cc, 8 * hh + cc] = 1.0
    return m


def _sel_mat(h, c, hp):
    return np.ascontiguousarray(np.transpose(_pad_mat(h, c, hp)))


_BSUM8 = _head_sum(8, 64)
_BBC8 = _head_bcast(8, 64)
_BSUM6 = _head_sum(6, 48)
_BBC6 = _head_bcast(6, 48)
_PAD2 = _pad_mat(8, 6, 8)
_SEL2 = _sel_mat(8, 6, 8)
_PAD3 = _pad_mat(5, 5, 6)
_SEL3 = _sel_mat(5, 5, 6)
_Z80 = np.zeros((ROWS_T, 80), np.float32)
_Z64 = np.zeros((ROWS_T, 64), np.float32)


def kernel(x, edge_index, W1, g0, b0, Wl1, Wr1, att1, g1, b1, W2, c2, Wl2,
           Wr2, att2, g2, bb2, W3, c3, Wl3, Wr3, att3, g3, bb3, Wp, cp):
    idt = jnp.int32
    loops = jnp.arange(N, dtype=idt)
    src = jnp.concatenate([edge_index[0].astype(idt), loops])
    dst = jnp.concatenate([edge_index[1].astype(idt), loops])
    npad = EP - ETOT
    srcg = jnp.concatenate([src, jnp.zeros((npad,), idt)]).reshape(
        NW, STEPS, CH)
    dstg = jnp.concatenate([dst, jnp.zeros((npad,), idt)]).reshape(
        NW, STEPS, CH)
    dsts = jnp.concatenate([dst, jnp.full((npad,), N, idt)]).reshape(
        NW, STEPS, CH)

    att1p = att1.reshape(1, 64)
    att2p = jnp.concatenate([att2, jnp.zeros((8, 2), _f32)],
                            axis=1).reshape(1, 64)
    att3p = jnp.concatenate(
        [jnp.concatenate([att3, jnp.zeros((5, 3), _f32)], axis=1),
         jnp.zeros((1, 8), _f32)], axis=0).reshape(1, 48)

    xl1, xr1 = _tc_dense1(x, W1, g0, b0, Wl1, Wr1)
    gxl1, gxr1 = _make_sc_gather(64)(xl1, xr1, srcg, dstg)
    s1 = _tc_edge(gxl1, gxr1, att1p, _BSUM8, _BBC8, 80)
    p1 = _make_sc_scatter(80)(s1, dsts, _Z80)

    xl2, xr2 = _tc_dense2(p1, g1, b1, W2, c2, Wl2, Wr2, _BBC8, _PAD2)
    gxl2, gxr2 = _make_sc_gather(64)(xl2, xr2, srcg, dstg)
    s2 = _tc_edge(gxl2, gxr2, att2p, _BSUM8, _BBC8, 80)
    p2 = _make_sc_scatter(80)(s2, dsts, _Z80)

    xl3, xr3 = _tc_dense3(p2, _SEL2, _BBC8, g2, bb2, W3, c3, Wl3, Wr3, _PAD3)
    gxl3, gxr3 = _make_sc_gather(48)(xl3, xr3, srcg, dstg)
    s3 = _tc_edge(gxl3, gxr3, att3p, _BSUM6, _BBC6, 64)
    p3 = _make_sc_scatter(64)(s3, dsts, _Z64)

    return _tc_dense4(p3, _SEL3, _BBC6, g3, bb3, Wp, cp)


# speedup vs baseline: 36.0877x; 1.0047x over previous
import functools

import jax
import jax.numpy as jnp
import numpy as np
from jax import lax
from jax.experimental import pallas as pl
from jax.experimental.pallas import tpu as pltpu
from jax.experimental.pallas import tpu_sc as plsc

N = 10000
E = 320000
ETOT = E + N
NW = 32
CH = 128
STEPS = 81
EW = STEPS * CH
EP = NW * EW
EBLK = 10368
NACC = 10240
ROWS_T = NACC // 16

_f32 = jnp.float32


def _mesh():
    return plsc.VectorSubcoreMesh(core_axis_name="c", subcore_axis_name="s",
                                  num_cores=2, num_subcores=16)


_SC_PARAMS = pltpu.CompilerParams(use_tc_tiling_on_sc=False)


@functools.lru_cache(maxsize=None)
def _make_sc_gather(hcp):
    @functools.partial(
        pl.kernel,
        out_type=(
            jax.ShapeDtypeStruct((EP, hcp), _f32),
            jax.ShapeDtypeStruct((EP, hcp), _f32),
        ),
        mesh=_mesh(),
        scratch_types=[
            pltpu.VMEM((STEPS, CH), jnp.int32),
            pltpu.VMEM((STEPS, CH), jnp.int32),
            pltpu.VMEM((CH, hcp), _f32),
            pltpu.VMEM((CH, hcp), _f32),
            pltpu.VMEM((CH, hcp), _f32),
            pltpu.VMEM((CH, hcp), _f32),
            pltpu.SemaphoreType.DMA,
            pltpu.SemaphoreType.DMA,
        ],
        compiler_params=_SC_PARAMS,
    )
    def k(xl_hbm, xr_hbm, src_hbm, dst_hbm, gxl_hbm, gxr_hbm,
          sidx, didx, lbuf0, rbuf0, lbuf1, rbuf1, lsem, rsem):
        wid = lax.axis_index("s") * 2 + lax.axis_index("c")
        base = wid * EW
        pltpu.sync_copy(src_hbm.at[wid], sidx)
        pltpu.sync_copy(dst_hbm.at[wid], didx)
        lbufs = (lbuf0, lbuf1)
        rbufs = (rbuf0, rbuf1)

        pltpu.async_copy(xl_hbm.at[sidx.at[0]], lbuf0, lsem)
        pltpu.async_copy(xr_hbm.at[didx.at[0]], rbuf0, rsem)

        def body(i, carry):
            for b in range(2):
                g = 2 * i + b
                nb = 1 - b

                @pl.when(g + 1 < STEPS)
                def _():
                    pltpu.async_copy(xl_hbm.at[sidx.at[g + 1]], lbufs[nb],
                                     lsem)
                    pltpu.async_copy(xr_hbm.at[didx.at[g + 1]], rbufs[nb],
                                     rsem)

                pltpu.make_async_copy(xl_hbm.at[sidx.at[g]], lbufs[b],
                                      lsem).wait()
                pltpu.make_async_copy(xr_hbm.at[didx.at[g]], rbufs[b],
                                      rsem).wait()
                pltpu.sync_copy(lbufs[b], gxl_hbm.at[pl.ds(base + g * CH,
                                                           CH)])
                pltpu.sync_copy(rbufs[b], gxr_hbm.at[pl.ds(base + g * CH,
                                                           CH)])
            return carry

        lax.fori_loop(0, (STEPS - 1) // 2, body, 0)
        g = STEPS - 1
        pltpu.make_async_copy(xl_hbm.at[sidx.at[g]], lbuf0, lsem).wait()
        pltpu.make_async_copy(xr_hbm.at[didx.at[g]], rbuf0, rsem).wait()
        pltpu.sync_copy(lbuf0, gxl_hbm.at[pl.ds(base + g * CH, CH)])
        pltpu.sync_copy(rbuf0, gxr_hbm.at[pl.ds(base + g * CH, CH)])

    return k


@functools.lru_cache(maxsize=None)
def _make_sc_scatter(w):
    @functools.partial(
        pl.kernel,
        out_type=jax.ShapeDtypeStruct((2, NACC, w), _f32),
        mesh=_mesh(),
        scratch_types=[
            pltpu.VMEM((STEPS, CH), jnp.int32),
            pltpu.VMEM((CH, w), _f32),
            pltpu.VMEM((CH, w), _f32),
            pltpu.VMEM_SHARED((NACC, w), _f32),
            pltpu.SemaphoreType.DMA,
        ],
        compiler_params=_SC_PARAMS,
    )
    def k(s_hbm, dst_hbm, zero_hbm, out_hbm, didx, sbuf0, sbuf1, acc, ssem):
        c = lax.axis_index("c")
        s = lax.axis_index("s")
        wid = s * 2 + c
        base = wid * EW
        pltpu.sync_copy(zero_hbm, acc.at[pl.ds(s * ROWS_T, ROWS_T)])
        pltpu.sync_copy(dst_hbm.at[wid], didx)
        plsc.subcore_barrier()
        sbufs = (sbuf0, sbuf1)

        pltpu.async_copy(s_hbm.at[pl.ds(base, CH)], sbuf0, ssem)

        def body(i, carry):
            for b in range(2):
                g = 2 * i + b
                nb = 1 - b

                @pl.when(g + 1 < STEPS)
                def _():
                    pltpu.async_copy(
                        s_hbm.at[pl.ds(base + (g + 1) * CH, CH)],
                        sbufs[nb], ssem)

                pltpu.make_async_copy(s_hbm.at[pl.ds(base + g * CH, CH)],
                                      sbufs[b], ssem).wait()
                pltpu.sync_copy(sbufs[b], acc.at[didx.at[g]], add=True)
            return carry

        lax.fori_loop(0, (STEPS - 1) // 2, body, 0)
        g = STEPS - 1
        pltpu.make_async_copy(s_hbm.at[pl.ds(base + g * CH, CH)],
                              sbuf0, ssem).wait()
        pltpu.sync_copy(sbuf0, acc.at[didx.at[g]], add=True)
        plsc.subcore_barrier()
        pltpu.sync_copy(acc.at[pl.ds(s * ROWS_T, ROWS_T)],
                        out_hbm.at[c, pl.ds(s * ROWS_T, ROWS_T)])

    return k


def _elu(x):
    return jnp.where(x > 0, x, jnp.exp(jnp.minimum(x, 0.0)) - 1.0)


def _bn(x, g, b):
    m = jnp.mean(x, axis=0, keepdims=True)
    v = jnp.mean((x - m) ** 2, axis=0, keepdims=True)
    return (x - m) / jnp.sqrt(v + 1e-5) * g + b


def _call_tc(body, out_shapes, args):
    return pl.pallas_call(
        body,
        out_shape=out_shapes,
    )(*args)


def _tc_dense1(x, W1, g0, b0, Wl1, Wr1):
    def body(x_r, w1_r, g0_r, b0_r, wl_r, wr_r, xl_o, xr_o):
        h = jnp.dot(x_r[...], w1_r[...], preferred_element_type=_f32)
        h = _bn(_elu(h), g0_r[...], b0_r[...])
        xl_o[...] = jnp.dot(h, wl_r[...], preferred_element_type=_f32)
        xr_o[...] = jnp.dot(h, wr_r[...], preferred_element_type=_f32)

    return _call_tc(
        body,
        (jax.ShapeDtypeStruct((N, 64), _f32),
         jax.ShapeDtypeStruct((N, 64), _f32)),
        (x, W1, g0.reshape(1, -1), b0.reshape(1, -1), Wl1, Wr1),
    )


def _tc_edge(gxl, gxr, attp, bsum, bbc, w):
    hcp = gxl.shape[1]
    hp = bsum.shape[1]

    def body(xl_r, xr_r, att_r, bs_r, bb_r, s_o):
        e = xl_r[...] + xr_r[...]
        e = jnp.where(e > 0, e, 0.2 * e)
        alpha = jnp.dot(e * att_r[...], bs_r[...],
                        preferred_element_type=_f32, precision=jax.lax.Precision.HIGHEST)
        ex = jnp.exp(alpha)
        msg = jnp.dot(ex, bb_r[...], preferred_element_type=_f32, precision=jax.lax.Precision.HIGHEST) * xl_r[...]
        s_o[...] = jnp.concatenate(
            [msg, ex, jnp.zeros((EBLK, w - hcp - hp), _f32)], axis=1)

    grid = EP // EBLK
    return pl.pallas_call(
        body,
        grid=(grid,),
        in_specs=[
            pl.BlockSpec((EBLK, hcp), lambda i: (i, 0)),
            pl.BlockSpec((EBLK, hcp), lambda i: (i, 0)),
            pl.BlockSpec((1, hcp), lambda i: (0, 0)),
            pl.BlockSpec((hcp, hp), lambda i: (0, 0)),
            pl.BlockSpec((hp, hcp), lambda i: (0, 0)),
        ],
        out_specs=pl.BlockSpec((EBLK, w), lambda i: (i, 0)),
        out_shape=jax.ShapeDtypeStruct((EP, w), _f32),
    )(gxl, gxr, attp, bsum, bbc)


def _tc_dense2(parts, g1, b1, W2, c2, Wl2, Wr2, bbc8, pad2):
    def body2(p_r, g1_r, b1_r, w2_r, c2_r, wl_r, wr_r, bb_r, pad_r, xl_o, xr_o):
        p = p_r[0] + p_r[1]
        num = p[:N, :64]
        den = p[:N, 64:72] + 1e-16
        a = num / jnp.dot(den, bb_r[...], preferred_element_type=_f32, precision=jax.lax.Precision.HIGHEST)
        h = _bn(a, g1_r[...], b1_r[...])
        h = _elu(jnp.dot(h, w2_r[...], preferred_element_type=_f32)
                 + c2_r[...])
        xl = jnp.dot(h, wl_r[...], preferred_element_type=_f32)
        xr = jnp.dot(h, wr_r[...], preferred_element_type=_f32)
        xl_o[...] = jnp.dot(xl, pad_r[...], preferred_element_type=_f32, precision=jax.lax.Precision.HIGHEST)
        xr_o[...] = jnp.dot(xr, pad_r[...], preferred_element_type=_f32, precision=jax.lax.Precision.HIGHEST)

    return _call_tc(
        body2,
        (jax.ShapeDtypeStruct((N, 64), _f32),
         jax.ShapeDtypeStruct((N, 64), _f32)),
        (parts, g1.reshape(1, -1), b1.reshape(1, -1), W2,
         c2.reshape(1, -1), Wl2, Wr2, bbc8, pad2),
    )


def _tc_dense3(parts, sel2, bbc8, g2, bb2, W3, c3, Wl3, Wr3, pad3):
    def body(p_r, sel_r, bb_r, g2_r, b2_r, w3_r, c3_r, wl_r, wr_r, pad_r,
             xl_o, xr_o):
        p = p_r[0] + p_r[1]
        num = p[:N, :64]
        den = p[:N, 64:72] + 1e-16
        a = num / jnp.dot(den, bb_r[...], preferred_element_type=_f32, precision=jax.lax.Precision.HIGHEST)
        g48 = jnp.dot(a, sel_r[...], preferred_element_type=_f32, precision=jax.lax.Precision.HIGHEST)
        h = _bn(g48, g2_r[...], b2_r[...])
        h = _elu(jnp.dot(h, w3_r[...], preferred_element_type=_f32)
                 + c3_r[...])
        xl = jnp.dot(h, wl_r[...], preferred_element_type=_f32)
        xr = jnp.dot(h, wr_r[...], preferred_element_type=_f32)
        xl_o[...] = jnp.dot(xl, pad_r[...], preferred_element_type=_f32, precision=jax.lax.Precision.HIGHEST)
        xr_o[...] = jnp.dot(xr, pad_r[...], preferred_element_type=_f32, precision=jax.lax.Precision.HIGHEST)

    return _call_tc(
        body,
        (jax.ShapeDtypeStruct((N, 48), _f32),
         jax.ShapeDtypeStruct((N, 48), _f32)),
        (parts, sel2, bbc8, g2.reshape(1, -1), bb2.reshape(1, -1), W3,
         c3.reshape(1, -1), Wl3, Wr3, pad3),
    )


def _tc_dense4(parts, sel3, bbc6, g3, bb3, Wp, cp):
    def body(p_r, sel_r, bb_r, g3_r, b3_r, wp_r, cp_r, out_o):
        p = p_r[0] + p_r[1]
        num = p[:N, :48]
        den = p[:N, 48:54] + 1e-16
        a = num / jnp.dot(den, bb_r[...], preferred_element_type=_f32, precision=jax.lax.Precision.HIGHEST)
        g25 = jnp.dot(a, sel_r[...], preferred_element_type=_f32, precision=jax.lax.Precision.HIGHEST)
        h = _bn(g25, g3_r[...], b3_r[...])
        out_o[...] = jnp.dot(h, wp_r[...], preferred_element_type=_f32) \
            + cp_r[...]

    return _call_tc(
        body,
        jax.ShapeDtypeStruct((N, 2), _f32),
        (parts, sel3, bbc6, g3.reshape(1, -1), bb3.reshape(1, -1), Wp,
         cp.reshape(1, -1)),
    )


def _head_sum(hp, hcp):
    m = np.zeros((hcp, hp), np.float32)
    for h in range(hp):
        m[8 * h:8 * h + 8, h] = 1.0
    return m


def _head_bcast(hp, hcp):
    return np.ascontiguousarray(np.transpose(_head_sum(hp, hcp)))


def _pad_mat(h, c, hp):
    m = np.zeros((h * c, hp * 8), np.float32)
    for hh in range(h):
        for cc in range(c):
            m[hh * c + cc, 8 * hh + cc] = 1.0
    return m


def _sel_mat(h, c, hp):
    return np.ascontiguousarray(np.transpose(_pad_mat(h, c, hp)))


_BSUM8 = _head_sum(8, 64)
_BBC8 = _head_bcast(8, 64)
_BSUM6 = _head_sum(6, 48)
_BBC6 = _head_bcast(6, 48)
_PAD2 = _pad_mat(8, 6, 8)
_SEL2 = _sel_mat(8, 6, 8)
_PAD3 = _pad_mat(5, 5, 6)
_SEL3 = _sel_mat(5, 5, 6)
_Z80 = np.zeros((ROWS_T, 80), np.float32)
_Z64 = np.zeros((ROWS_T, 64), np.float32)


def kernel(x, edge_index, W1, g0, b0, Wl1, Wr1, att1, g1, b1, W2, c2, Wl2,
           Wr2, att2, g2, bb2, W3, c3, Wl3, Wr3, att3, g3, bb3, Wp, cp):
    idt = jnp.int32
    loops = jnp.arange(N, dtype=idt)
    src = jnp.concatenate([edge_index[0].astype(idt), loops])
    dst = jnp.concatenate([edge_index[1].astype(idt), loops])
    npad = EP - ETOT
    srcg = jnp.concatenate([src, jnp.zeros((npad,), idt)]).reshape(
        NW, STEPS, CH)
    dstg = jnp.concatenate([dst, jnp.zeros((npad,), idt)]).reshape(
        NW, STEPS, CH)
    dsts = jnp.concatenate([dst, jnp.full((npad,), N, idt)]).reshape(
        NW, STEPS, CH)

    att1p = att1.reshape(1, 64)
    att2p = jnp.concatenate([att2, jnp.zeros((8, 2), _f32)],
                            axis=1).reshape(1, 64)
    att3p = jnp.concatenate(
        [jnp.concatenate([att3, jnp.zeros((5, 3), _f32)], axis=1),
         jnp.zeros((1, 8), _f32)], axis=0).reshape(1, 48)

    xl1, xr1 = _tc_dense1(x, W1, g0, b0, Wl1, Wr1)
    gxl1, gxr1 = _make_sc_gather(64)(xl1, xr1, srcg, dstg)
    s1 = _tc_edge(gxl1, gxr1, att1p, _BSUM8, _BBC8, 80)
    p1 = _make_sc_scatter(80)(s1, dsts, _Z80)

    xl2, xr2 = _tc_dense2(p1, g1, b1, W2, c2, Wl2, Wr2, _BBC8, _PAD2)
    gxl2, gxr2 = _make_sc_gather(64)(xl2, xr2, srcg, dstg)
    s2 = _tc_edge(gxl2, gxr2, att2p, _BSUM8, _BBC8, 80)
    p2 = _make_sc_scatter(80)(s2, dsts, _Z80)

    xl3, xr3 = _tc_dense3(p2, _SEL2, _BBC8, g2, bb2, W3, c3, Wl3, Wr3, _PAD3)
    gxl3, gxr3 = _make_sc_gather(48)(xl3, xr3, srcg, dstg)
    s3 = _tc_edge(gxl3, gxr3, att3p, _BSUM6, _BBC6, 64)
    p3 = _make_sc_scatter(64)(s3, dsts, _Z64)

    return _tc_dense4(p3, _SEL3, _BBC6, g3, bb3, Wp, cp)
